# baseline (speedup 1.0000x reference)
import functools

import numpy as np
import jax
import jax.numpy as jnp
from jax import lax
from jax.experimental import pallas as pl
from jax.experimental.pallas import tpu as pltpu
from jax.experimental.pallas import tpu_sc as plsc

_NEG_INF = float("-inf")
_EPS = 1e-4
_LOG_EPS = float(np.log(np.float32(_EPS)))
_TINY = np.float32(np.finfo(np.float32).tiny)

_DN_T = (((1,), (1,)), ((), ()))
_DN = (((1,), (0,)), ((), ()))


def _mm(x, w, dn):
    return lax.dot_general(x, w, dimension_numbers=dn,
                           preferred_element_type=jnp.float32)


def _threefry_gumbel(k1, k2, idx):
    rots = ((13, 15, 26, 6), (17, 29, 16, 24))
    ks0, ks1 = k1, k2
    ks2 = k1 ^ k2 ^ np.uint32(0x1BD11BDA)
    x0 = jnp.zeros_like(idx) + ks0
    x1 = idx + ks1
    kseq = (ks1, ks2, ks0)
    for i in range(5):
        for r in rots[i % 2]:
            x0 = x0 + x1
            x1 = lax.shift_left(x1, np.uint32(r)) | lax.shift_right_logical(
                x1, np.uint32(32 - r))
            x1 = x0 ^ x1
        x0 = x0 + kseq[i % 3]
        x1 = x1 + kseq[(i + 1) % 3] + np.uint32(i + 1)
    bits = x0 ^ x1
    fb = lax.shift_right_logical(bits, np.uint32(9)) | np.uint32(0x3F800000)
    u = lax.bitcast_convert_type(fb, jnp.float32) - np.float32(1.0)
    u = jnp.maximum(_TINY, u * np.float32(1.0 - _TINY) + _TINY)
    return -jnp.log(-jnp.log(u))


def _qhead_kernel(x_ref, w0, b0, w1, b1, w2, b2, wf, bf, wq, out_ref):
    o = out_ref.shape[1]
    q = x_ref[...]
    for w, b in ((w0, b0), (w1, b1), (w2, b2)):
        q = jnp.maximum(_mm(q, w[...], _DN_T) + b[...], 0.0)
    q = _mm(q, wf[...], _DN_T) + bf[...]
    out_ref[...] = _mm(jnp.maximum(q, 0.0), wq[:, :o], _DN_T)


def _kmain_kernel(x_ref, bi_ref, psb_ref, w0, b0, w1, b1, w2, b2, wf, bf,
                  wg, f0b, f1w, f1b, qf_ref,
                  logits_ref, z_ref, c_ref,
                  plog_ref, m_ref, s_ref):
    t = pl.program_id(0)
    nb = qf_ref.shape[0]
    tt = x_ref.shape[0]
    seg_iota = lax.broadcasted_iota(jnp.int32, (tt, nb), 1)

    valid = t > 0
    plog = plog_ref[0, :]
    psb = psb_ref[0, 0, :]
    mask = psb[:, None] == seg_iota
    tile_max = jnp.where(
        valid, jnp.max(jnp.where(mask, plog[:, None], _NEG_INF), axis=0),
        _NEG_INF)
    tile_cnt = jnp.where(valid, jnp.sum(mask.astype(jnp.float32), axis=0), 0.0)
    m_prev = jnp.where(t <= 1, _NEG_INF, m_ref[0, :])
    s_prev = jnp.where(t <= 1, 0.0, s_ref[0, :])
    c_prev = jnp.where(t <= 1, 0.0, c_ref[0, :])
    m_new = jnp.maximum(m_prev, tile_max)
    alpha = jnp.where(m_prev == _NEG_INF, 0.0, jnp.exp(m_prev - m_new))
    tile_sum = jnp.where(
        valid & mask, jnp.exp(plog[:, None] - m_new[None, :]), 0.0
    ).sum(axis=0)
    s_new = s_prev * alpha + tile_sum
    m_ref[0, :] = m_new
    s_ref[0, :] = s_new
    c_ref[0, :] = c_prev + tile_cnt
    z_ref[0, :] = m_new + jnp.log(s_new)

    h = x_ref[...]
    for w, b in ((w0, b0), (w1, b1), (w2, b2)):
        h = jnp.maximum(_mm(h, w[...], _DN_T) + b[...], 0.0)
    k = _mm(h, wf[...], _DN_T) + bf[...]
    kg = _mm(jnp.maximum(k, 0.0), wg[...], _DN_T)

    bi = bi_ref[0, 0, :]
    oh = (bi[:, None] == seg_iota).astype(jnp.float32)
    qe = _mm(oh, qf_ref[...], _DN)

    pre = jnp.maximum(kg + qe + f0b[...], 0.0)
    logit = jnp.sum(pre * f1w[...], axis=1) + f1b[0, 0]
    logits_ref[0, 0, :] = logit
    plog_ref[0, :] = logit


def _sc_probs(logits_flat, sb, z_flat):
    n = logits_flat.shape[0]
    nbz = z_flat.shape[0]
    info = plsc.get_sparse_core_info()
    nw = info.num_cores * info.num_subcores
    lanes = info.num_lanes
    chunk = n // nw
    mesh = plsc.VectorSubcoreMesh(core_axis_name="c", subcore_axis_name="s")

    @functools.partial(
        pl.kernel, mesh=mesh,
        out_type=jax.ShapeDtypeStruct((n,), jnp.float32),
        scratch_types=[
            pltpu.VMEM((chunk,), jnp.float32),
            pltpu.VMEM((chunk,), jnp.int32),
            pltpu.VMEM((nbz + lanes,), jnp.float32),
            pltpu.VMEM((chunk,), jnp.float32),
        ],
    )
    def k(lg_hbm, sb_hbm, z_hbm, out_hbm, lg_v, sb_v, z_s, pr_v):
        wid = lax.axis_index("s") * info.num_cores + lax.axis_index("c")
        base = wid * chunk
        pltpu.sync_copy(lg_hbm.at[pl.ds(base, chunk)], lg_v)
        pltpu.sync_copy(sb_hbm.at[pl.ds(base, chunk)], sb_v)
        pltpu.sync_copy(z_hbm, z_s)
        lo = sb_v[pl.ds(0, lanes)][0]
        hi = sb_v[pl.ds(chunk - lanes, lanes)][lanes - 1]
        for i in range(chunk // lanes):
            sl = pl.ds(i * lanes, lanes)
            sbv = sb_v[sl]

            def seg_body(s, zz):
                return jnp.where(sbv == s, z_s[pl.ds(s, lanes)][0], zz)

            zz = lax.fori_loop(lo, hi + 1, seg_body,
                               jnp.zeros((lanes,), jnp.float32))
            pr_v[sl] = jnp.exp(lg_v[sl] - zz)
        pltpu.sync_copy(pr_v, out_hbm.at[pl.ds(base, chunk)])

    return k(logits_flat, sb,
             jnp.pad(z_flat, (0, lanes)))


def _sample_kernel(logits_ref, sb_ref, kd_ref, z_ref, c_ref,
                   act_ref, alp_ref,
                   bv_ref, bix_ref, blp_ref):
    t = pl.program_id(0)
    nt = pl.num_programs(0)
    nb = z_ref.shape[1]
    tt = logits_ref.shape[2]
    n = nt * tt

    logit = logits_ref[0, 0, :]
    sb = sb_ref[0, 0, :]
    seg_iota = lax.broadcasted_iota(jnp.int32, (tt, nb), 1)
    mask = sb[:, None] == seg_iota
    z_e = jnp.sum(jnp.where(mask, z_ref[0, :][None, :], 0.0), axis=1)
    lp = logit - z_e

    col = t * tt + lax.broadcasted_iota(jnp.int32, (tt,), 0)
    idx = (sb * n + col).astype(jnp.uint32)
    g = _threefry_gumbel(kd_ref[0, 0], kd_ref[0, 1], idx)

    score = jnp.where(lp > _LOG_EPS, g + lp, _NEG_INF)
    tbest = jnp.max(jnp.where(mask, score[:, None], _NEG_INF), axis=0)
    row_iota = lax.broadcasted_iota(jnp.int32, (tt, nb), 0)
    hit = mask & (score[:, None] == tbest[None, :])
    targ = jnp.min(jnp.where(hit, row_iota, tt), axis=0)
    first = mask & (row_iota == targ[None, :])
    t_lp = jnp.sum(jnp.where(first, lp[:, None], 0.0), axis=0)

    @pl.when(t == 0)
    def _init():
        bv_ref[...] = jnp.full(bv_ref.shape, _NEG_INF, jnp.float32)
        bix_ref[...] = jnp.zeros(bix_ref.shape, jnp.int32)
        blp_ref[...] = jnp.full(blp_ref.shape, lp[0], jnp.float32)

    upd = tbest > bv_ref[0, :]
    bv_ref[0, :] = jnp.where(upd, tbest, bv_ref[0, :])
    bix_ref[0, :] = jnp.where(upd, t * tt + targ, bix_ref[0, :])
    blp_ref[0, :] = jnp.where(upd, t_lp, blp_ref[0, :])

    @pl.when(t == nt - 1)
    def _fin():
        c = c_ref[0, :]
        i = lax.broadcasted_iota(jnp.int32, (nb, nb), 0)
        j = lax.broadcasted_iota(jnp.int32, (nb, nb), 1)
        shifts = jnp.sum(jnp.where(i < j, c[:, None], 0.0), axis=0)
        act_ref[0, :] = bix_ref[0, :] - shifts.astype(jnp.int32)
        alp_ref[0, :] = blp_ref[0, :]


def kernel(states, candidates, batch_idx, Qw0, Qb0, Qw1, Qb1, Qw2, Qb2,
           Kw0, Kb0, Kw1, Kb1, Kw2, Kb2, Qfw, Qfb, Kfw, Kfb,
           F0w, F0b, F1w, F1b):
    nb, d = states.shape
    n = candidates.shape[0]
    o = Qfw.shape[0]
    h = Qw0.shape[0]
    tt = 512
    nt = n // tt
    tts = 2048
    nts = n // tts

    f32 = jnp.float32
    row = lambda v: v.reshape(1, -1).astype(f32)

    bi = batch_idx.astype(jnp.int32)
    trans = (bi[1:] != bi[:-1]).astype(jnp.int32)
    sb = jnp.cumsum(jnp.concatenate([jnp.zeros((1,), jnp.int32), trans]))
    bi3 = bi.reshape(nt, 1, tt)
    sb3 = sb.reshape(nt, 1, tt)
    sb3s = sb.reshape(nts, 1, tts)

    kd = jax.random.key_data(jax.random.key(1234)).reshape(1, 2)

    full = lambda shp: pl.BlockSpec(shp, lambda i: (0,) * len(shp))

    qf = pl.pallas_call(
        _qhead_kernel,
        out_shape=jax.ShapeDtypeStruct((nb, o), f32),
    )(states, Qw0, row(Qb0), Qw1, row(Qb1), Qw2, row(Qb2),
      Qfw, row(Qfb), F0w)

    tile3 = pl.BlockSpec((1, 1, tt), lambda i: (i, 0, 0))
    last = nt - 1
    tile3c = pl.BlockSpec((1, 1, tt), lambda i: (jnp.minimum(i, last), 0, 0))
    tile3p = pl.BlockSpec((1, 1, tt), lambda i: (jnp.maximum(i - 1, 0), 0, 0))

    logits3, z, c = pl.pallas_call(
        _kmain_kernel,
        grid=(nt + 1,),
        in_specs=[
            pl.BlockSpec((tt, d), lambda i: (jnp.minimum(i, nt - 1), 0)),
            tile3c, tile3p,
            full((h, d)), full((1, h)),
            full((h, h)), full((1, h)),
            full((h, h)), full((1, h)),
            full((o, h)), full((1, o)),
            pl.BlockSpec((o, o), lambda i: (0, 1)),
            full((1, o)), full((1, o)), full((1, 1)),
            full((nb, o)),
        ],
        out_specs=[
            tile3c,
            full((1, nb)), full((1, nb)),
        ],
        out_shape=[
            jax.ShapeDtypeStruct((nt, 1, tt), f32),
            jax.ShapeDtypeStruct((1, nb), f32),
            jax.ShapeDtypeStruct((1, nb), f32),
        ],
        scratch_shapes=[
            pltpu.VMEM((1, tt), f32),
            pltpu.VMEM((1, nb), f32),
            pltpu.VMEM((1, nb), f32),
        ],
        compiler_params=pltpu.CompilerParams(
            vmem_limit_bytes=110 * 1024 * 1024),
    )(candidates, bi3, sb3,
      Kw0, row(Kb0), Kw1, row(Kb1), Kw2, row(Kb2),
      Kfw, row(Kfb), F0w, row(F0b), row(F1w), F1b.reshape(1, 1), qf)

    act, alp = pl.pallas_call(
        _sample_kernel,
        grid=(nts,),
        in_specs=[
            pl.BlockSpec((1, 1, tts), lambda i: (i, 0, 0)),
            pl.BlockSpec((1, 1, tts), lambda i: (i, 0, 0)),
            full((1, 2)),
            full((1, nb)), full((1, nb)),
        ],
        out_specs=[
            full((1, nb)), full((1, nb)),
        ],
        out_shape=[
            jax.ShapeDtypeStruct((1, nb), jnp.int32),
            jax.ShapeDtypeStruct((1, nb), f32),
        ],
        scratch_shapes=[
            pltpu.VMEM((1, nb), f32),
            pltpu.VMEM((1, nb), jnp.int32),
            pltpu.VMEM((1, nb), f32),
        ],
    )(logits3.reshape(nts, 1, tts), sb3s, kd, z, c)

    probs = _sc_probs(logits3.reshape(n), sb, z.reshape(nb))

    return probs, alp.reshape(nb), act.reshape(nb)


# speedup vs baseline: 1.0508x; 1.0508x over previous
import numpy as np
import jax
import jax.numpy as jnp
from jax import lax
from jax.experimental import pallas as pl
from jax.experimental.pallas import tpu as pltpu

_NEG_INF = float("-inf")
_EPS = 1e-4
_TINY = np.float32(np.finfo(np.float32).tiny)

_DN_T = (((1,), (1,)), ((), ()))
_DN = (((1,), (0,)), ((), ()))


def _mm(x, w, dn):
    return lax.dot_general(x, w, dimension_numbers=dn,
                           preferred_element_type=jnp.float32)


def _threefry_gumbel(k1, k2, idx):
    rots = ((13, 15, 26, 6), (17, 29, 16, 24))
    ks0, ks1 = k1, k2
    ks2 = k1 ^ k2 ^ np.uint32(0x1BD11BDA)
    x0 = jnp.zeros_like(idx) + ks0
    x1 = idx + ks1
    kseq = (ks1, ks2, ks0)
    for i in range(5):
        for r in rots[i % 2]:
            x0 = x0 + x1
            x1 = lax.shift_left(x1, np.uint32(r)) | lax.shift_right_logical(
                x1, np.uint32(32 - r))
            x1 = x0 ^ x1
        x0 = x0 + kseq[i % 3]
        x1 = x1 + kseq[(i + 1) % 3] + np.uint32(i + 1)
    bits = x0 ^ x1
    fb = lax.shift_right_logical(bits, np.uint32(9)) | np.uint32(0x3F800000)
    u = lax.bitcast_convert_type(fb, jnp.float32) - np.float32(1.0)
    u = jnp.maximum(_TINY, u * np.float32(1.0 - _TINY) + _TINY)
    return -jnp.log(-jnp.log(u))


def _qhead_kernel(x_ref, w0, b0, w1, b1, w2, b2, wf, bf, wq, out_ref):
    o = out_ref.shape[1]
    q = x_ref[...]
    for w, b in ((w0, b0), (w1, b1), (w2, b2)):
        q = jnp.maximum(_mm(q, w[...], _DN_T) + b[...], 0.0)
    q = _mm(q, wf[...], _DN_T) + bf[...]
    out_ref[...] = _mm(jnp.maximum(q, 0.0), wq[:, :o], _DN_T)


def _kmain_kernel(x_ref, bi_ref, psb_ref, w0, b0, w1, b1, w2, b2, wf, bf,
                  wg, f0b, f1w, f1b, qf_ref,
                  logits_ref, z_ref, c_ref,
                  plog_ref, m_ref, s_ref):
    t = pl.program_id(0)
    nb = qf_ref.shape[0]
    tt = x_ref.shape[0]
    seg_iota = lax.broadcasted_iota(jnp.int32, (tt, nb), 1)

    valid = t > 0
    plog = plog_ref[0, :]
    psb = psb_ref[0, 0, :]
    mask = psb[:, None] == seg_iota
    tile_max = jnp.where(
        valid, jnp.max(jnp.where(mask, plog[:, None], _NEG_INF), axis=0),
        _NEG_INF)
    tile_cnt = jnp.where(valid, jnp.sum(mask.astype(jnp.float32), axis=0), 0.0)
    m_prev = jnp.where(t <= 1, _NEG_INF, m_ref[0, :])
    s_prev = jnp.where(t <= 1, 0.0, s_ref[0, :])
    c_prev = jnp.where(t <= 1, 0.0, c_ref[0, :])
    m_new = jnp.maximum(m_prev, tile_max)
    alpha = jnp.where(m_prev == _NEG_INF, 0.0, jnp.exp(m_prev - m_new))
    tile_sum = jnp.where(
        valid & mask, jnp.exp(plog[:, None] - m_new[None, :]), 0.0
    ).sum(axis=0)
    s_new = s_prev * alpha + tile_sum
    m_ref[0, :] = m_new
    s_ref[0, :] = s_new
    c_ref[0, :] = c_prev + tile_cnt
    z_ref[0, :] = m_new + jnp.log(s_new)

    h = x_ref[...]
    for w, b in ((w0, b0), (w1, b1), (w2, b2)):
        h = jnp.maximum(_mm(h, w[...], _DN_T) + b[...], 0.0)
    k = _mm(h, wf[...], _DN_T) + bf[...]
    kg = _mm(jnp.maximum(k, 0.0), wg[...], _DN_T)

    bi = bi_ref[0, 0, :]
    oh = (bi[:, None] == seg_iota).astype(jnp.float32)
    qe = _mm(oh, qf_ref[...], _DN)

    pre = jnp.maximum(kg + qe + f0b[...], 0.0)
    logit = jnp.sum(pre * f1w[...], axis=1) + f1b[0, 0]
    logits_ref[0, 0, :] = logit
    plog_ref[0, :] = logit


def _sample_kernel(logits_ref, sb_ref, kd_ref, z_ref, c_ref,
                   probs_ref, act_ref, alp_ref,
                   bv_ref, bix_ref, blp_ref):
    t = pl.program_id(0)
    nt = pl.num_programs(0)
    nb = z_ref.shape[1]
    tt = logits_ref.shape[2]
    n = nt * tt

    logit = logits_ref[0, 0, :]
    sb = sb_ref[0, 0, :]
    seg_iota = lax.broadcasted_iota(jnp.int32, (tt, nb), 1)
    mask = sb[:, None] == seg_iota
    z_e = jnp.sum(jnp.where(mask, z_ref[0, :][None, :], 0.0), axis=1)
    lp = logit - z_e
    probs = jnp.exp(lp)
    probs_ref[0, 0, :] = probs

    col = t * tt + lax.broadcasted_iota(jnp.int32, (tt,), 0)
    idx = (sb * n + col).astype(jnp.uint32)
    g = _threefry_gumbel(kd_ref[0, 0], kd_ref[0, 1], idx)

    score = jnp.where(probs > _EPS, g + lp, _NEG_INF)
    tbest = jnp.max(jnp.where(mask, score[:, None], _NEG_INF), axis=0)
    row_iota = lax.broadcasted_iota(jnp.int32, (tt, nb), 0)
    hit = mask & (score[:, None] == tbest[None, :])
    targ = jnp.min(jnp.where(hit, row_iota, tt), axis=0)
    first = mask & (row_iota == targ[None, :])
    t_lp = jnp.sum(jnp.where(first, lp[:, None], 0.0), axis=0)

    @pl.when(t == 0)
    def _init():
        bv_ref[...] = jnp.full(bv_ref.shape, _NEG_INF, jnp.float32)
        bix_ref[...] = jnp.zeros(bix_ref.shape, jnp.int32)
        blp_ref[...] = jnp.full(blp_ref.shape, lp[0], jnp.float32)

    upd = tbest > bv_ref[0, :]
    bv_ref[0, :] = jnp.where(upd, tbest, bv_ref[0, :])
    bix_ref[0, :] = jnp.where(upd, t * tt + targ, bix_ref[0, :])
    blp_ref[0, :] = jnp.where(upd, t_lp, blp_ref[0, :])

    @pl.when(t == nt - 1)
    def _fin():
        c = c_ref[0, :]
        i = lax.broadcasted_iota(jnp.int32, (nb, nb), 0)
        j = lax.broadcasted_iota(jnp.int32, (nb, nb), 1)
        shifts = jnp.sum(jnp.where(i < j, c[:, None], 0.0), axis=0)
        act_ref[0, :] = bix_ref[0, :] - shifts.astype(jnp.int32)
        alp_ref[0, :] = blp_ref[0, :]


def kernel(states, candidates, batch_idx, Qw0, Qb0, Qw1, Qb1, Qw2, Qb2,
           Kw0, Kb0, Kw1, Kb1, Kw2, Kb2, Qfw, Qfb, Kfw, Kfb,
           F0w, F0b, F1w, F1b):
    nb, d = states.shape
    n = candidates.shape[0]
    o = Qfw.shape[0]
    h = Qw0.shape[0]
    tt = 512
    nt = n // tt
    tts = 2048
    nts = n // tts

    f32 = jnp.float32
    row = lambda v: v.reshape(1, -1).astype(f32)

    bi = batch_idx.astype(jnp.int32)
    trans = (bi[1:] != bi[:-1]).astype(jnp.int32)
    sb = jnp.cumsum(jnp.concatenate([jnp.zeros((1,), jnp.int32), trans]))
    bi3 = bi.reshape(nt, 1, tt)
    sb3 = sb.reshape(nt, 1, tt)
    sb3s = sb.reshape(nts, 1, tts)

    kd = jax.random.key_data(jax.random.key(1234)).reshape(1, 2)

    full = lambda shp: pl.BlockSpec(shp, lambda i: (0,) * len(shp))

    qf = pl.pallas_call(
        _qhead_kernel,
        out_shape=jax.ShapeDtypeStruct((nb, o), f32),
    )(states, Qw0, row(Qb0), Qw1, row(Qb1), Qw2, row(Qb2),
      Qfw, row(Qfb), F0w)

    tile3 = pl.BlockSpec((1, 1, tt), lambda i: (i, 0, 0))
    last = nt - 1
    tile3c = pl.BlockSpec((1, 1, tt), lambda i: (jnp.minimum(i, last), 0, 0))
    tile3p = pl.BlockSpec((1, 1, tt), lambda i: (jnp.maximum(i - 1, 0), 0, 0))

    logits3, z, c = pl.pallas_call(
        _kmain_kernel,
        grid=(nt + 1,),
        in_specs=[
            pl.BlockSpec((tt, d), lambda i: (jnp.minimum(i, nt - 1), 0)),
            tile3c, tile3p,
            full((h, d)), full((1, h)),
            full((h, h)), full((1, h)),
            full((h, h)), full((1, h)),
            full((o, h)), full((1, o)),
            pl.BlockSpec((o, o), lambda i: (0, 1)),
            full((1, o)), full((1, o)), full((1, 1)),
            full((nb, o)),
        ],
        out_specs=[
            tile3c,
            full((1, nb)), full((1, nb)),
        ],
        out_shape=[
            jax.ShapeDtypeStruct((nt, 1, tt), f32),
            jax.ShapeDtypeStruct((1, nb), f32),
            jax.ShapeDtypeStruct((1, nb), f32),
        ],
        scratch_shapes=[
            pltpu.VMEM((1, tt), f32),
            pltpu.VMEM((1, nb), f32),
            pltpu.VMEM((1, nb), f32),
        ],
        compiler_params=pltpu.CompilerParams(
            vmem_limit_bytes=110 * 1024 * 1024),
    )(candidates, bi3, sb3,
      Kw0, row(Kb0), Kw1, row(Kb1), Kw2, row(Kb2),
      Kfw, row(Kfb), F0w, row(F0b), row(F1w), F1b.reshape(1, 1), qf)

    tile3s = pl.BlockSpec((1, 1, tts), lambda i: (i, 0, 0))
    probs3, act, alp = pl.pallas_call(
        _sample_kernel,
        grid=(nts,),
        in_specs=[
            tile3s, tile3s,
            full((1, 2)),
            full((1, nb)), full((1, nb)),
        ],
        out_specs=[
            tile3s,
            full((1, nb)), full((1, nb)),
        ],
        out_shape=[
            jax.ShapeDtypeStruct((nts, 1, tts), f32),
            jax.ShapeDtypeStruct((1, nb), jnp.int32),
            jax.ShapeDtypeStruct((1, nb), f32),
        ],
        scratch_shapes=[
            pltpu.VMEM((1, nb), f32),
            pltpu.VMEM((1, nb), jnp.int32),
            pltpu.VMEM((1, nb), f32),
        ],
    )(logits3.reshape(nts, 1, tts), sb3s, kd, z, c)

    return probs3.reshape(n), alp.reshape(nb), act.reshape(nb)
